# baseline (speedup 1.0000x reference)
import functools

import jax
import jax.numpy as jnp
from jax import lax
from jax.experimental import pallas as pl
from jax.experimental.pallas import tpu as pltpu
from jax.experimental.pallas import tpu_sc as plsc


def kernel(indices, table):
    B, H = indices.shape
    V, D = table.shape
    N = B * H

    info = plsc.get_sparse_core_info()
    NC, NS = info.num_cores, info.num_subcores
    NW = NC * NS
    per_w = N // NW

    SUB = 128
    C = 640
    n_sub = C // SUB
    n_chunks = per_w // C

    idx3 = indices.reshape(NW * n_chunks, n_sub, SUB)

    mesh = plsc.VectorSubcoreMesh(core_axis_name="c", subcore_axis_name="s")

    @functools.partial(
        pl.kernel,
        mesh=mesh,
        out_type=jax.ShapeDtypeStruct((N, D), jnp.float32),
        scratch_types=[
            pltpu.VMEM((n_sub, SUB), jnp.int32),
            pltpu.VMEM((C, D), jnp.float32),
            pltpu.SemaphoreType.DMA,
        ],
        compiler_params=pltpu.CompilerParams(use_tc_tiling_on_sc=False),
    )
    def gather_kernel(idx_hbm, table_hbm, out_hbm, idx_v, rows_v, sem):
        wid = lax.axis_index("s") * NC + lax.axis_index("c")
        base = wid * per_w

        def body(g, carry):
            chunk_id = wid * n_chunks + g
            pltpu.sync_copy(idx_hbm.at[chunk_id], idx_v)
            copies = []
            for j in range(n_sub):
                copies.append(
                    pltpu.async_copy(
                        table_hbm.at[idx_v.at[j]],
                        rows_v.at[pl.ds(j * SUB, SUB)],
                        sem,
                    )
                )
            for c in copies:
                c.wait()
            pltpu.sync_copy(rows_v, out_hbm.at[pl.ds(base + g * C, C)])
            return carry

        lax.fori_loop(0, n_chunks, body, 0)

    out = gather_kernel(idx3, table)
    return out.reshape(B, H, D)


# speedup vs baseline: 1.0279x; 1.0279x over previous
import functools

import jax
import jax.numpy as jnp
from jax import lax
from jax.experimental import pallas as pl
from jax.experimental.pallas import tpu as pltpu
from jax.experimental.pallas import tpu_sc as plsc


def kernel(indices, table):
    B, H = indices.shape
    V, D = table.shape
    N = B * H

    info = plsc.get_sparse_core_info()
    NC, NS = info.num_cores, info.num_subcores
    NW = NC * NS
    per_w = N // NW

    SUB = 128
    C = 640
    NB = 2
    n_sub = C // SUB
    n_chunks = per_w // C
    T = n_chunks // NB

    idx3 = indices.reshape(NW * n_chunks, n_sub, SUB)

    mesh = plsc.VectorSubcoreMesh(core_axis_name="c", subcore_axis_name="s")

    @functools.partial(
        pl.kernel,
        mesh=mesh,
        out_type=jax.ShapeDtypeStruct((N, D), jnp.float32),
        scratch_types=[
            pltpu.VMEM((NB, n_sub, SUB), jnp.int32),
            pltpu.VMEM((NB, C, D), jnp.float32),
            [pltpu.SemaphoreType.DMA] * NB,
            [pltpu.SemaphoreType.DMA] * NB,
            [pltpu.SemaphoreType.DMA] * NB,
        ],
        compiler_params=pltpu.CompilerParams(use_tc_tiling_on_sc=False),
    )
    def gather_kernel(idx_hbm, table_hbm, out_hbm, idx_v, rows_v, isems, gsems, ssems):
        wid = lax.axis_index("s") * NC + lax.axis_index("c")
        base = wid * per_w
        cbase = wid * n_chunks

        def fire_idx(chunk_id, b):
            pltpu.async_copy(idx_hbm.at[chunk_id], idx_v.at[b], isems[b])

        def wait_idx(b):
            pltpu.make_async_copy(idx_hbm.at[0], idx_v.at[b], isems[b]).wait()

        def fire_gathers(b):
            for j in range(n_sub):
                pltpu.async_copy(
                    table_hbm.at[idx_v.at[b, j]],
                    rows_v.at[b, pl.ds(j * SUB, SUB)],
                    gsems[b],
                )

        def wait_gathers(b):
            pltpu.make_async_copy(
                out_hbm.at[pl.ds(0, C)], rows_v.at[b], gsems[b]
            ).wait()

        def fire_scatter(row0, b):
            pltpu.async_copy(rows_v.at[b], out_hbm.at[pl.ds(row0, C)], ssems[b])

        def wait_scatter(b):
            pltpu.make_async_copy(
                rows_v.at[b], out_hbm.at[pl.ds(0, C)], ssems[b]
            ).wait()

        for b in range(NB):
            fire_idx(cbase + b, b)

        def body(t, carry):
            g0 = t * NB
            for b in range(NB):
                @pl.when(t > 0)
                def _():
                    wait_scatter(b)

                wait_idx(b)
                fire_gathers(b)
            for b in range(NB):
                wait_gathers(b)
                fire_scatter(base + (g0 + b) * C, b)

                @pl.when(t < T - 1)
                def _():
                    fire_idx(cbase + g0 + NB + b, b)

            return carry

        lax.fori_loop(0, T, body, 0)
        for b in range(NB):
            wait_scatter(b)

    out = gather_kernel(idx3, table)
    return out.reshape(B, H, D)
